# Initial kernel scaffold; baseline (speedup 1.0000x reference)
import jax
import jax.numpy as jnp
from jax.experimental import pallas as pl


def kernel(x, tokens_embed, positions_embed):
    raise NotImplementedError("write your pallas kernel here")


# speedup vs baseline: 1.0141x; 1.0141x over previous
import functools

import jax
import jax.numpy as jnp
from jax import lax
from jax.experimental import pallas as pl
from jax.experimental.pallas import tpu as pltpu
from jax.experimental.pallas import tpu_sc as plsc

_LANES = 16


@functools.partial(jax.jit, static_argnums=(3, 4))
def _emb_lookup(x_flat, tokens_embed, positions_embed, n_workers, chunk):
    n, = x_flat.shape
    _, d = tokens_embed.shape
    seq_len, _ = positions_embed.shape
    per_w = n // n_workers
    n_chunks = per_w // chunk

    mesh = plsc.VectorSubcoreMesh(core_axis_name="c", subcore_axis_name="s")

    @functools.partial(
        pl.kernel,
        out_type=jax.ShapeDtypeStruct((n, d), jnp.float32),
        mesh=mesh,
        scratch_types=[
            pltpu.VMEM((per_w,), jnp.int32),
            pltpu.VMEM((chunk, d), jnp.float32),
            pltpu.VMEM((chunk, d), jnp.float32),
            pltpu.SemaphoreType.DMA,
            pltpu.SemaphoreType.DMA,
        ],
    )
    def emb_kernel(x_hbm, tok_hbm, pos_hbm, out_hbm, idx_v, tokbuf, posbuf,
                   gsem, psem):
        wid = lax.axis_index("s") * 2 + lax.axis_index("c")
        base = wid * per_w
        s0 = lax.rem(base, seq_len)
        pltpu.sync_copy(x_hbm.at[pl.ds(base, per_w)], idx_v)
        for c in range(n_chunks):
            gcp = pltpu.async_copy(
                tok_hbm.at[idx_v.at[pl.ds(c * chunk, chunk)]], tokbuf, gsem)
            pcp = pltpu.async_copy(
                pos_hbm.at[pl.ds(s0 + c * chunk, chunk)], posbuf, psem)
            gcp.wait()
            pcp.wait()

            @plsc.parallel_loop(0, chunk, unroll=2)
            def _(r):
                for j in range(d // _LANES):
                    sl = pl.ds(j * _LANES, _LANES)
                    tokbuf[r, sl] += posbuf[r, sl]

            pltpu.sync_copy(tokbuf, out_hbm.at[pl.ds(base + c * chunk, chunk)])

    return emb_kernel(x_flat, tokens_embed, positions_embed)


def kernel(x, tokens_embed, positions_embed):
    b, s = x.shape
    _, d = tokens_embed.shape
    x_flat = x.reshape(b * s).astype(jnp.int32)
    out = _emb_lookup(x_flat, tokens_embed, positions_embed, 32, 64)
    return out.reshape(b, s, d)


# speedup vs baseline: 1.1519x; 1.1358x over previous
import functools

import jax
import jax.numpy as jnp
from jax import lax
from jax.experimental import pallas as pl
from jax.experimental.pallas import tpu as pltpu
from jax.experimental.pallas import tpu_sc as plsc

_LANES = 16


@functools.partial(jax.jit, static_argnums=(3, 4))
def _emb_lookup(x_flat, tokens_embed, positions_embed, n_workers, chunk):
    n, = x_flat.shape
    _, d = tokens_embed.shape
    seq_len, _ = positions_embed.shape
    per_w = n // n_workers
    n_chunks = per_w // chunk

    mesh = plsc.VectorSubcoreMesh(core_axis_name="c", subcore_axis_name="s")

    @functools.partial(
        pl.kernel,
        out_type=jax.ShapeDtypeStruct((n, d), jnp.float32),
        mesh=mesh,
        scratch_types=[
            pltpu.VMEM((per_w,), jnp.int32),
            pltpu.VMEM((2, chunk, d), jnp.float32),
            pltpu.VMEM((2, chunk, d), jnp.float32),
            pltpu.SemaphoreType.DMA,
            pltpu.SemaphoreType.DMA,
            pltpu.SemaphoreType.DMA,
            pltpu.SemaphoreType.DMA,
            pltpu.SemaphoreType.DMA,
            pltpu.SemaphoreType.DMA,
        ],
    )
    def emb_kernel(x_hbm, tok_hbm, pos_hbm, out_hbm, idx_v, tokbuf, posbuf,
                   g0, g1, p0, p1, o0, o1):
        gsem = (g0, g1)
        psem = (p0, p1)
        osem = (o0, o1)
        wid = lax.axis_index("s") * 2 + lax.axis_index("c")
        base = wid * per_w
        s0 = lax.rem(base, seq_len)
        pltpu.sync_copy(x_hbm.at[pl.ds(base, per_w)], idx_v)

        def issue(c):
            b = c % 2
            pltpu.async_copy(
                tok_hbm.at[idx_v.at[pl.ds(c * chunk, chunk)]],
                tokbuf.at[b], gsem[b])
            pltpu.async_copy(
                pos_hbm.at[pl.ds(s0 + c * chunk, chunk)],
                posbuf.at[b], psem[b])

        issue(0)
        out_cps = [None, None]
        for c in range(n_chunks):
            b = c % 2
            if c + 1 < n_chunks:
                nb = (c + 1) % 2
                if out_cps[nb] is not None:
                    out_cps[nb].wait()
                    out_cps[nb] = None
                issue(c + 1)
            pltpu.make_async_copy(
                tok_hbm.at[idx_v.at[pl.ds(c * chunk, chunk)]],
                tokbuf.at[b], gsem[b]).wait()
            pltpu.make_async_copy(
                pos_hbm.at[pl.ds(s0 + c * chunk, chunk)],
                posbuf.at[b], psem[b]).wait()

            @plsc.parallel_loop(0, chunk, unroll=2)
            def _(r):
                for j in range(d // _LANES):
                    sl = pl.ds(j * _LANES, _LANES)
                    tokbuf[b, r, sl] += posbuf[b, r, sl]

            out_cps[b] = pltpu.async_copy(
                tokbuf.at[b], out_hbm.at[pl.ds(base + c * chunk, chunk)],
                osem[b])
        for cp in out_cps:
            if cp is not None:
                cp.wait()

    return emb_kernel(x_flat, tokens_embed, positions_embed)


def kernel(x, tokens_embed, positions_embed):
    b, s = x.shape
    _, d = tokens_embed.shape
    x_flat = x.reshape(b * s).astype(jnp.int32)
    out = _emb_lookup(x_flat, tokens_embed, positions_embed, 32, 32)
    return out.reshape(b, s, d)


# speedup vs baseline: 1.2264x; 1.0647x over previous
import functools

import jax
import jax.numpy as jnp
from jax import lax
from jax.experimental import pallas as pl
from jax.experimental.pallas import tpu as pltpu
from jax.experimental.pallas import tpu_sc as plsc

_LANES = 16
_NW = 32


@jax.jit
def _emb_lookup(x, tokens_embed, positions_embed):
    batch, seq_len = x.shape
    _, d = tokens_embed.shape
    s_blk = seq_len // _NW
    half = s_blk // 2
    n_chunks = 2 * batch

    mesh = plsc.VectorSubcoreMesh(core_axis_name="c", subcore_axis_name="s")

    @functools.partial(
        pl.kernel,
        out_type=jax.ShapeDtypeStruct((batch, seq_len, d), jnp.float32),
        mesh=mesh,
        scratch_types=[
            pltpu.VMEM((batch, s_blk), jnp.int32),
            pltpu.VMEM((2, half, d), jnp.float32),
            pltpu.VMEM((s_blk, d), jnp.float32),
            pltpu.SemaphoreType.DMA,
            pltpu.SemaphoreType.DMA,
            pltpu.SemaphoreType.DMA,
            pltpu.SemaphoreType.DMA,
            pltpu.SemaphoreType.DMA,
        ],
    )
    def emb_kernel(x_hbm, tok_hbm, pos_hbm, out_hbm, idx_v, tokbuf, posbuf,
                   g0, g1, o0, o1, psem):
        gsem = (g0, g1)
        osem = (o0, o1)
        wid = lax.axis_index("s") * 2 + lax.axis_index("c")
        s0 = wid * s_blk
        pcp = pltpu.async_copy(pos_hbm.at[pl.ds(s0, s_blk)], posbuf, psem)
        for b in range(batch):
            pltpu.sync_copy(x_hbm.at[b, pl.ds(s0, s_blk)], idx_v.at[b])

        def issue(c):
            b, h = divmod(c, 2)
            pltpu.async_copy(
                tok_hbm.at[idx_v.at[b, pl.ds(h * half, half)]],
                tokbuf.at[c % 2], gsem[c % 2])

        issue(0)
        pcp.wait()
        out_cps = [None, None]
        for c in range(n_chunks):
            cb = c % 2
            b, h = divmod(c, 2)
            if c + 1 < n_chunks:
                if out_cps[1 - cb] is not None:
                    out_cps[1 - cb].wait()
                    out_cps[1 - cb] = None
                issue(c + 1)
            pltpu.make_async_copy(
                tok_hbm.at[idx_v.at[b, pl.ds(h * half, half)]],
                tokbuf.at[cb], gsem[cb]).wait()

            @plsc.parallel_loop(0, half, unroll=2)
            def _(r):
                for j in range(d // _LANES):
                    sl = pl.ds(j * _LANES, _LANES)
                    tokbuf[cb, r, sl] += posbuf[h * half + r, sl]

            out_cps[cb] = pltpu.async_copy(
                tokbuf.at[cb],
                out_hbm.at[b, pl.ds(s0 + h * half, half)], osem[cb])
        for cp in out_cps:
            if cp is not None:
                cp.wait()

    return emb_kernel(x, tokens_embed, positions_embed)


def kernel(x, tokens_embed, positions_embed):
    return _emb_lookup(x.astype(jnp.int32), tokens_embed, positions_embed)


# speedup vs baseline: 1.2641x; 1.0307x over previous
import functools

import jax
import jax.numpy as jnp
from jax import lax
from jax.experimental import pallas as pl
from jax.experimental.pallas import tpu as pltpu
from jax.experimental.pallas import tpu_sc as plsc

_LANES = 16
_NW = 32


@jax.jit
def _emb_lookup(x, tokens_embed, positions_embed):
    batch, seq_len = x.shape
    _, d = tokens_embed.shape
    s_blk = seq_len // _NW
    half = s_blk // 2
    n_chunks = 2 * batch

    mesh = plsc.VectorSubcoreMesh(core_axis_name="c", subcore_axis_name="s")

    @functools.partial(
        pl.kernel,
        out_type=jax.ShapeDtypeStruct((batch, seq_len, d), jnp.float32),
        mesh=mesh,
        scratch_types=[
            pltpu.VMEM((batch, s_blk), jnp.int32),
            pltpu.VMEM((3, half, d), jnp.float32),
            pltpu.VMEM((s_blk, d), jnp.float32),
            pltpu.SemaphoreType.DMA,
            pltpu.SemaphoreType.DMA,
            pltpu.SemaphoreType.DMA,
            pltpu.SemaphoreType.DMA,
            pltpu.SemaphoreType.DMA,
            pltpu.SemaphoreType.DMA,
            pltpu.SemaphoreType.DMA,
        ],
    )
    def emb_kernel(x_hbm, tok_hbm, pos_hbm, out_hbm, idx_v, tokbuf, posbuf,
                   g0, g1, g2, o0, o1, o2, psem):
        gsem = (g0, g1, g2)
        osem = (o0, o1, o2)
        wid = lax.axis_index("s") * 2 + lax.axis_index("c")
        s0 = wid * s_blk
        pcp = pltpu.async_copy(pos_hbm.at[pl.ds(s0, s_blk)], posbuf, psem)
        for b in range(batch):
            pltpu.sync_copy(x_hbm.at[b, pl.ds(s0, s_blk)], idx_v.at[b])

        def issue_gather(c):
            b, h = divmod(c, 2)
            cb = c % 3
            pltpu.async_copy(
                tok_hbm.at[idx_v.at[b, pl.ds(h * half, half)]],
                tokbuf.at[cb], gsem[cb])

        issue_gather(0)
        pcp.wait()
        out_cps = [None, None, None]
        for c in range(n_chunks):
            cb = c % 3
            b, h = divmod(c, 2)
            if c + 1 < n_chunks:
                nb = (c + 1) % 3
                if out_cps[nb] is not None:
                    out_cps[nb].wait()
                    out_cps[nb] = None
                issue_gather(c + 1)
            pltpu.make_async_copy(
                tok_hbm.at[idx_v.at[b, pl.ds(h * half, half)]],
                tokbuf.at[cb], gsem[cb]).wait()

            @plsc.parallel_loop(0, half, unroll=2)
            def _(r):
                for j in range(d // _LANES):
                    sl = pl.ds(j * _LANES, _LANES)
                    tokbuf[cb, r, sl] += posbuf[h * half + r, sl]

            out_cps[cb] = pltpu.async_copy(
                tokbuf.at[cb], out_hbm.at[b, pl.ds(s0 + h * half, half)],
                osem[cb])
        for cp in out_cps:
            if cp is not None:
                cp.wait()

    return emb_kernel(x, tokens_embed, positions_embed)


def kernel(x, tokens_embed, positions_embed):
    return _emb_lookup(x.astype(jnp.int32), tokens_embed, positions_embed)


# speedup vs baseline: 1.5590x; 1.2333x over previous
import functools

import jax
import jax.numpy as jnp
from jax import lax
from jax.experimental import pallas as pl
from jax.experimental.pallas import tpu as pltpu
from jax.experimental.pallas import tpu_sc as plsc

_LANES = 16
_NW = 32


@jax.jit
def _emb_lookup(x, tokens_embed, positions_embed):
    batch, seq_len = x.shape
    _, d = tokens_embed.shape
    s_blk = seq_len // _NW
    half = s_blk // 2
    n_chunks = 2 * batch

    mesh = plsc.VectorSubcoreMesh(core_axis_name="c", subcore_axis_name="s")

    @functools.partial(
        pl.kernel,
        out_type=jax.ShapeDtypeStruct((batch, seq_len, d), jnp.float32),
        mesh=mesh,
        scratch_types=[
            pltpu.VMEM((batch, s_blk), jnp.int32),
            pltpu.VMEM((3, half, d), jnp.float32),
            pltpu.VMEM((s_blk, d), jnp.float32),
            pltpu.SemaphoreType.DMA,
            pltpu.SemaphoreType.DMA,
            pltpu.SemaphoreType.DMA,
            pltpu.SemaphoreType.DMA,
            pltpu.SemaphoreType.DMA,
            pltpu.SemaphoreType.DMA,
            pltpu.SemaphoreType.DMA,
            pltpu.SemaphoreType.DMA,
        ],
    )
    def emb_kernel(x_hbm, tok_hbm, pos_hbm, out_hbm, idx_v, tokbuf, posbuf,
                   g0, g1, g2, o0, o1, o2, psem, isem):
        gsem = (g0, g1, g2)
        osem = (o0, o1, o2)
        wid = lax.axis_index("s") * 2 + lax.axis_index("c")
        s0 = wid * s_blk
        icps = [pltpu.async_copy(x_hbm.at[b, pl.ds(s0, s_blk)],
                                 idx_v.at[b], isem)
                for b in range(batch)]
        pcp = pltpu.async_copy(pos_hbm.at[pl.ds(s0, s_blk)], posbuf, psem)

        def for_slot(cb, fn):
            for k in range(3):
                @pl.when(cb == k)
                def _():
                    fn(k)

        def issue_gather(c, cb):
            b = c // 2
            h = lax.rem(c, 2)
            src = tok_hbm.at[idx_v.at[b, pl.ds(h * half, half)]]
            for_slot(cb, lambda k: pltpu.async_copy(
                src, tokbuf.at[k], gsem[k]))

        def wait_gather(cb):
            for_slot(cb, lambda k: pltpu.make_async_copy(
                tok_hbm.at[idx_v.at[0, pl.ds(0, half)]],
                tokbuf.at[k], gsem[k]).wait())

        def issue_drain(c, cb):
            b = c // 2
            h = lax.rem(c, 2)
            dst = out_hbm.at[b, pl.ds(s0 + h * half, half)]
            for_slot(cb, lambda k: pltpu.async_copy(
                tokbuf.at[k], dst, osem[k]))

        def wait_drain(cb):
            for_slot(cb, lambda k: pltpu.make_async_copy(
                tokbuf.at[k], out_hbm.at[0, pl.ds(s0, half)],
                osem[k]).wait())

        for icp in icps:
            icp.wait()
        issue_gather(jnp.int32(0), jnp.int32(0))
        pcp.wait()

        def body(c, _):
            cb = lax.rem(c, 3)
            h = lax.rem(c, 2)

            @pl.when(c + 1 < n_chunks)
            def _():
                nb = lax.rem(c + 1, 3)

                @pl.when(c >= 2)
                def _():
                    wait_drain(nb)
                issue_gather(c + 1, nb)

            wait_gather(cb)

            @plsc.parallel_loop(0, half)
            def _(r):
                for j in range(d // _LANES):
                    sl = pl.ds(j * _LANES, _LANES)
                    tokbuf[cb, r, sl] += posbuf[h * half + r, sl]

            issue_drain(c, cb)
            return None

        lax.fori_loop(0, n_chunks, body, None)
        wait_drain(jnp.int32((n_chunks - 2) % 3))
        wait_drain(jnp.int32((n_chunks - 1) % 3))

    return emb_kernel(x, tokens_embed, positions_embed)


def kernel(x, tokens_embed, positions_embed):
    return _emb_lookup(x.astype(jnp.int32), tokens_embed, positions_embed)


# speedup vs baseline: 1.5780x; 1.0122x over previous
import functools

import jax
import jax.numpy as jnp
from jax import lax
from jax.experimental import pallas as pl
from jax.experimental.pallas import tpu as pltpu
from jax.experimental.pallas import tpu_sc as plsc

_LANES = 16
_NW = 32
_CHUNK = 16
_RING = 4
_AHEAD = 2


@jax.jit
def _emb_lookup(x, tokens_embed, positions_embed):
    batch, seq_len = x.shape
    _, d = tokens_embed.shape
    s_blk = seq_len // _NW
    per_b = s_blk // _CHUNK
    n_chunks = batch * per_b

    mesh = plsc.VectorSubcoreMesh(core_axis_name="c", subcore_axis_name="s")

    @functools.partial(
        pl.kernel,
        out_type=jax.ShapeDtypeStruct((batch, seq_len, d), jnp.float32),
        mesh=mesh,
        scratch_types=[
            pltpu.VMEM((batch, s_blk), jnp.int32),
            pltpu.VMEM((_RING, _CHUNK, d), jnp.float32),
            pltpu.VMEM((s_blk, d), jnp.float32),
        ] + [pltpu.SemaphoreType.DMA] * (2 * _RING + 2),
    )
    def emb_kernel(x_hbm, tok_hbm, pos_hbm, out_hbm, idx_v, tokbuf, posbuf,
                   *sems):
        gsem = sems[:_RING]
        osem = sems[_RING:2 * _RING]
        psem, isem = sems[2 * _RING:]
        wid = lax.axis_index("s") * 2 + lax.axis_index("c")
        s0 = wid * s_blk
        icps = [pltpu.async_copy(x_hbm.at[b, pl.ds(s0, s_blk)],
                                 idx_v.at[b], isem)
                for b in range(batch)]
        pcp = pltpu.async_copy(pos_hbm.at[pl.ds(s0, s_blk)], posbuf, psem)

        def for_slot(cb, fn):
            for k in range(_RING):
                @pl.when(cb == k)
                def _():
                    fn(k)

        def issue_gather(c, cb):
            b = c // per_b
            q = lax.rem(c, per_b)
            src = tok_hbm.at[idx_v.at[b, pl.ds(q * _CHUNK, _CHUNK)]]
            for_slot(cb, lambda k: pltpu.async_copy(
                src, tokbuf.at[k], gsem[k]))

        def wait_gather(cb):
            for_slot(cb, lambda k: pltpu.make_async_copy(
                tok_hbm.at[idx_v.at[0, pl.ds(0, _CHUNK)]],
                tokbuf.at[k], gsem[k]).wait())

        def issue_drain(c, cb):
            b = c // per_b
            q = lax.rem(c, per_b)
            dst = out_hbm.at[b, pl.ds(s0 + q * _CHUNK, _CHUNK)]
            for_slot(cb, lambda k: pltpu.async_copy(
                tokbuf.at[k], dst, osem[k]))

        def wait_drain(cb):
            for_slot(cb, lambda k: pltpu.make_async_copy(
                tokbuf.at[k], out_hbm.at[0, pl.ds(s0, _CHUNK)],
                osem[k]).wait())

        for icp in icps:
            icp.wait()
        for c0 in range(_AHEAD):
            issue_gather(jnp.int32(c0), jnp.int32(c0))
        pcp.wait()

        def body(c, _):
            cb = lax.rem(c, _RING)
            q = lax.rem(c, per_b)

            @pl.when(c + _AHEAD < n_chunks)
            def _():
                nb = lax.rem(c + _AHEAD, _RING)

                @pl.when(c + _AHEAD >= _RING)
                def _():
                    wait_drain(nb)
                issue_gather(c + _AHEAD, nb)

            wait_gather(cb)

            @plsc.parallel_loop(0, _CHUNK)
            def _(r):
                for j in range(d // _LANES):
                    sl = pl.ds(j * _LANES, _LANES)
                    tokbuf[cb, r, sl] += posbuf[q * _CHUNK + r, sl]

            issue_drain(c, cb)
            return None

        lax.fori_loop(0, n_chunks, body, None)
        for c in range(n_chunks - _RING, n_chunks):
            wait_drain(jnp.int32(c % _RING))

    return emb_kernel(x, tokens_embed, positions_embed)


def kernel(x, tokens_embed, positions_embed):
    return _emb_lookup(x.astype(jnp.int32), tokens_embed, positions_embed)
